# baseline (speedup 1.0000x reference)
import dataclasses
import functools

import jax
import jax.numpy as jnp
from jax import lax
from jax.experimental import pallas as pl
from jax.experimental.pallas import tpu as pltpu
from jax.experimental.pallas import tpu_sc as plsc

_EPS = 1e-5
_B = 16384
_D = 32
_N = 1000000
_FULL_TILES = _N // 128
_MAIN = _FULL_TILES * 128
_TAIL = _N - _MAIN
_NC = 2
_NS = 16
_NW = _NC * _NS
_PPW = _B // _NW
_PCHUNK = 128
_NCHUNK = _PPW // _PCHUNK
_OPC = 8 * _PCHUNK
_EPC = _D * _PCHUNK
_L = 16


def _sc_distance_arg(sidx, didx, tabs, tail):
    mesh = plsc.VectorSubcoreMesh(core_axis_name="c", subcore_axis_name="s")
    cp = pltpu.CompilerParams()
    for field, val in (("needs_layout_passes", False),
                       ("use_tc_tiling_on_sc", False)):
        if field in pltpu.CompilerParams.__dataclass_fields__:
            cp = dataclasses.replace(cp, **{field: val})

    @functools.partial(
        pl.kernel,
        mesh=mesh,
        compiler_params=cp,
        out_type=jax.ShapeDtypeStruct((_B,), jnp.float32),
        scratch_types=[
            pltpu.VMEM((_PPW,), jnp.int32),
            pltpu.VMEM((_PPW,), jnp.int32),
            pltpu.VMEM((_TAIL * _D,), jnp.float32),
            pltpu.VMEM((2, _OPC), jnp.int32),
            pltpu.VMEM((2, _OPC), jnp.int32),
            pltpu.VMEM((2, _EPC), jnp.float32),
            pltpu.VMEM((2, _EPC), jnp.float32),
            pltpu.VMEM((_PPW,), jnp.float32),
            pltpu.SemaphoreType.DMA,
        ],
    )
    def sc_kernel(sidx_hbm, didx_hbm, t0, t1, t2, t3, tail_hbm, out_hbm,
                  sidx_v, didx_v, tail_v, soff_v, doff_v, su_v, dv_v, x_v,
                  sem):
        trefs = (t0, t1, t2, t3)
        wid = lax.axis_index("s") * _NC + lax.axis_index("c")
        base = wid * _PPW
        pltpu.sync_copy(sidx_hbm.at[pl.ds(base, _PPW)], sidx_v)
        pltpu.sync_copy(didx_hbm.at[pl.ds(base, _PPW)], didx_v)
        pltpu.sync_copy(tail_hbm, tail_v)

        def build_offsets(g, buf):
            @pl.loop(0, _PCHUNK // _L)
            def _(grp):
                sv = sidx_v[pl.ds(g * _PCHUNK + grp * _L, _L)]
                dv = didx_v[pl.ds(g * _PCHUNK + grp * _L, _L)]
                sv = jnp.minimum(sv, _MAIN - 1)
                dv = jnp.minimum(dv, _MAIN - 1)
                sb = lax.shift_right_logical(sv, 7) * 1024 + (sv & 127)
                db = lax.shift_right_logical(dv, 7) * 1024 + (dv & 127)
                for c_lo in range(8):
                    pos = c_lo * _PCHUNK + grp * _L
                    soff_v[buf, pl.ds(pos, _L)] = sb + c_lo * 128
                    doff_v[buf, pl.ds(pos, _L)] = db + c_lo * 128
        def fire_gathers(buf):
            copies = []
            for cg in range(4):
                for j in range(_OPC // 128):
                    src = pl.ds(j * 128, 128)
                    dst = pl.ds(cg * _OPC + j * 128, 128)
                    copies.append(pltpu.async_copy(
                        trefs[cg].at[soff_v.at[buf, src]],
                        su_v.at[buf, dst], sem))
                    copies.append(pltpu.async_copy(
                        trefs[cg].at[doff_v.at[buf, src]],
                        dv_v.at[buf, dst], sem))
            return copies

        def _emit_x(sq, un, vn, out_slice):
            un = jnp.minimum(jnp.maximum(un, 0.0), 1.0 - _EPS)
            vn = jnp.minimum(jnp.maximum(vn, 0.0), 1.0 - _EPS)
            x_v[out_slice] = 1.0 + 2.0 * sq / ((1.0 - un) * (1.0 - vn))

        def compute_chunk(g, buf):
            @pl.loop(0, _PCHUNK // _L)
            def _(grp):
                out_slice = pl.ds(g * _PCHUNK + grp * _L, _L)
                sq = un = vn = None
                for c in range(_D):
                    pos = c * _PCHUNK + grp * _L
                    u = su_v[buf, pl.ds(pos, _L)]
                    v = dv_v[buf, pl.ds(pos, _L)]
                    d = u - v
                    if c == 0:
                        sq, un, vn = d * d, u * u, v * v
                    else:
                        sq = d * d + sq
                        un = u * u + un
                        vn = v * v + vn
                _emit_x(sq, un, vn, out_slice)

                sv = sidx_v[out_slice]
                dvi = didx_v[out_slice]
                sm = sv >= _MAIN
                dm = dvi >= _MAIN
                any_tail = jnp.any(jnp.logical_or(sm, dm))

                @pl.when(any_tail)
                def _():
                    st = jnp.maximum(sv - _MAIN, 0) * _D
                    dt = jnp.maximum(dvi - _MAIN, 0) * _D
                    sq = un = vn = None
                    for c in range(_D):
                        pos = c * _PCHUNK + grp * _L
                        u = su_v[buf, pl.ds(pos, _L)]
                        v = dv_v[buf, pl.ds(pos, _L)]
                        u = jnp.where(
                            sm, plsc.load_gather(tail_v, [st + c]), u)
                        v = jnp.where(
                            dm, plsc.load_gather(tail_v, [dt + c]), v)
                        d = u - v
                        if c == 0:
                            sq, un, vn = d * d, u * u, v * v
                        else:
                            sq = d * d + sq
                            un = u * u + un
                            vn = v * v + vn
                    _emit_x(sq, un, vn, out_slice)

        build_offsets(0, 0)
        pending = fire_gathers(0)
        for g in range(_NCHUNK):
            if g + 1 < _NCHUNK:
                build_offsets(g + 1, (g + 1) % 2)
            for cpy in pending:
                cpy.wait()
            if g + 1 < _NCHUNK:
                pending = fire_gathers((g + 1) % 2)
            compute_chunk(g, g % 2)

        pltpu.sync_copy(x_v, out_hbm.at[pl.ds(base, _PPW)])

    return sc_kernel(sidx, didx, *tabs, tail)


def _tc_detile(emb_t):
    def body(in_ref, o0, o1, o2, o3, s0, s1, s2, s3):
        outs = (o0, o1, o2, o3)
        sems = (s0, s1, s2, s3)
        copies = []
        for cg in range(4):
            copies.append(pltpu.make_async_copy(
                in_ref.at[pl.ds(8 * cg, 8), pl.ds(0, _MAIN)],
                outs[cg], sems[cg]))
        for cpy in copies:
            cpy.start()
        for cpy in copies:
            cpy.wait()

    return pl.pallas_call(
        body,
        in_specs=[pl.BlockSpec(memory_space=pl.ANY)],
        out_specs=[pl.BlockSpec(memory_space=pl.ANY)] * 4,
        out_shape=[jax.ShapeDtypeStruct((8, _MAIN), jnp.float32)] * 4,
        scratch_shapes=[pltpu.SemaphoreType.DMA] * 4,
    )(emb_t)


def _tc_arccosh(x2d):
    def body(x_ref, o_ref):
        x = jnp.maximum(x_ref[...], 1.0 + _EPS)
        o_ref[...] = jnp.log(x + jnp.sqrt(x * x - 1.0))

    return pl.pallas_call(
        body,
        out_shape=jax.ShapeDtypeStruct(x2d.shape, jnp.float32),
    )(x2d)


@jax.jit
def kernel(input_triplet, emb_table):
    sidx = input_triplet[:, 0]
    didx = input_triplet[:, 1]
    groups = _tc_detile(emb_table.T)
    tabs = tuple(
        grp.reshape(8, _FULL_TILES, 128)
        .transpose(1, 0, 2)
        .reshape(-1)
        for grp in groups
    )
    tail = emb_table[_MAIN:, :].reshape(-1)
    x = _sc_distance_arg(sidx, didx, tabs, tail)
    dist = _tc_arccosh(x.reshape(_B // 128, 128))
    return dist.reshape(_B)


# speedup vs baseline: 26.9185x; 26.9185x over previous
import dataclasses
import functools

import jax
import jax.numpy as jnp
from jax import lax
from jax.experimental import pallas as pl
from jax.experimental.pallas import tpu as pltpu
from jax.experimental.pallas import tpu_sc as plsc

_EPS = 1e-5
_B = 16384
_D = 32
_N = 1000000
_FULL_TILES = _N // 128
_MAIN = _FULL_TILES * 128
_TAIL = _N - _MAIN
_NC = 2
_NS = 16
_NW = _NC * _NS
_PPW = _B // _NW
_PCHUNK = 128
_NCHUNK = _PPW // _PCHUNK
_OPC = 8 * _PCHUNK
_EPC = _D * _PCHUNK
_L = 16


def _sc_distance_arg(sidx, didx, tabs, tail):
    mesh = plsc.VectorSubcoreMesh(core_axis_name="c", subcore_axis_name="s")
    cp = pltpu.CompilerParams()
    for field, val in (("needs_layout_passes", False),
                       ("use_tc_tiling_on_sc", False)):
        if field in pltpu.CompilerParams.__dataclass_fields__:
            cp = dataclasses.replace(cp, **{field: val})

    @functools.partial(
        pl.kernel,
        mesh=mesh,
        compiler_params=cp,
        out_type=jax.ShapeDtypeStruct((_B,), jnp.float32),
        scratch_types=[
            pltpu.VMEM((_PPW,), jnp.int32),
            pltpu.VMEM((_PPW,), jnp.int32),
            pltpu.VMEM((_TAIL * _D,), jnp.float32),
            pltpu.VMEM((2, _OPC), jnp.int32),
            pltpu.VMEM((2, _OPC), jnp.int32),
            pltpu.VMEM((2, _EPC), jnp.float32),
            pltpu.VMEM((2, _EPC), jnp.float32),
            pltpu.VMEM((_PPW,), jnp.float32),
            pltpu.SemaphoreType.DMA,
        ],
    )
    def sc_kernel(sidx_hbm, didx_hbm, t0, t1, t2, t3, tail_hbm, out_hbm,
                  sidx_v, didx_v, tail_v, soff_v, doff_v, su_v, dv_v, x_v,
                  sem):
        trefs = (t0, t1, t2, t3)
        wid = lax.axis_index("s") * _NC + lax.axis_index("c")
        base = wid * _PPW
        pltpu.sync_copy(sidx_hbm.at[pl.ds(base, _PPW)], sidx_v)
        pltpu.sync_copy(didx_hbm.at[pl.ds(base, _PPW)], didx_v)
        pltpu.sync_copy(tail_hbm, tail_v)

        def build_offsets(g, buf):
            @pl.loop(0, _PCHUNK // _L)
            def _(grp):
                sv = sidx_v[pl.ds(g * _PCHUNK + grp * _L, _L)]
                dv = didx_v[pl.ds(g * _PCHUNK + grp * _L, _L)]
                sv = jnp.minimum(sv, _MAIN - 1)
                dv = jnp.minimum(dv, _MAIN - 1)
                sb = lax.shift_right_logical(sv, 7) * 1024 + (sv & 127)
                db = lax.shift_right_logical(dv, 7) * 1024 + (dv & 127)
                for c_lo in range(8):
                    pos = c_lo * _PCHUNK + grp * _L
                    soff_v[buf, pl.ds(pos, _L)] = sb + c_lo * 128
                    doff_v[buf, pl.ds(pos, _L)] = db + c_lo * 128
        def fire_gathers(buf):
            copies = []
            for cg in range(4):
                for j in range(_OPC // 128):
                    src = pl.ds(j * 128, 128)
                    dst = pl.ds(cg * _OPC + j * 128, 128)
                    copies.append(pltpu.async_copy(
                        trefs[cg].at[soff_v.at[buf, src]],
                        su_v.at[buf, dst], sem))
                    copies.append(pltpu.async_copy(
                        trefs[cg].at[doff_v.at[buf, src]],
                        dv_v.at[buf, dst], sem))
            return copies

        def _emit_x(sq, un, vn, out_slice):
            un = jnp.minimum(jnp.maximum(un, 0.0), 1.0 - _EPS)
            vn = jnp.minimum(jnp.maximum(vn, 0.0), 1.0 - _EPS)
            x_v[out_slice] = 1.0 + 2.0 * sq / ((1.0 - un) * (1.0 - vn))

        def compute_chunk(g, buf):
            @pl.loop(0, _PCHUNK // _L)
            def _(grp):
                out_slice = pl.ds(g * _PCHUNK + grp * _L, _L)
                sq = un = vn = None
                for c in range(_D):
                    pos = c * _PCHUNK + grp * _L
                    u = su_v[buf, pl.ds(pos, _L)]
                    v = dv_v[buf, pl.ds(pos, _L)]
                    d = u - v
                    if c == 0:
                        sq, un, vn = d * d, u * u, v * v
                    else:
                        sq = d * d + sq
                        un = u * u + un
                        vn = v * v + vn
                _emit_x(sq, un, vn, out_slice)

                sv = sidx_v[out_slice]
                dvi = didx_v[out_slice]
                sm = sv >= _MAIN
                dm = dvi >= _MAIN
                any_tail = jnp.any(jnp.logical_or(sm, dm))

                @pl.when(any_tail)
                def _():
                    st = jnp.maximum(sv - _MAIN, 0) * _D
                    dt = jnp.maximum(dvi - _MAIN, 0) * _D
                    sq = un = vn = None
                    for c in range(_D):
                        pos = c * _PCHUNK + grp * _L
                        u = su_v[buf, pl.ds(pos, _L)]
                        v = dv_v[buf, pl.ds(pos, _L)]
                        u = jnp.where(
                            sm, plsc.load_gather(tail_v, [st + c]), u)
                        v = jnp.where(
                            dm, plsc.load_gather(tail_v, [dt + c]), v)
                        d = u - v
                        if c == 0:
                            sq, un, vn = d * d, u * u, v * v
                        else:
                            sq = d * d + sq
                            un = u * u + un
                            vn = v * v + vn
                    _emit_x(sq, un, vn, out_slice)

        build_offsets(0, 0)
        pending = fire_gathers(0)
        for g in range(_NCHUNK):
            if g + 1 < _NCHUNK:
                build_offsets(g + 1, (g + 1) % 2)
            for cpy in pending:
                cpy.wait()
            if g + 1 < _NCHUNK:
                pending = fire_gathers((g + 1) % 2)
            compute_chunk(g, g % 2)

        pltpu.sync_copy(x_v, out_hbm.at[pl.ds(base, _PPW)])

    return sc_kernel(sidx, didx, *tabs, tail)


def _tc_arccosh(x2d):
    def body(x_ref, o_ref):
        x = jnp.maximum(x_ref[...], 1.0 + _EPS)
        o_ref[...] = jnp.log(x + jnp.sqrt(x * x - 1.0))

    return pl.pallas_call(
        body,
        out_shape=jax.ShapeDtypeStruct(x2d.shape, jnp.float32),
    )(x2d)


@jax.jit
def kernel(input_triplet, emb_table):
    sidx = input_triplet[:, 0]
    didx = input_triplet[:, 1]
    tabs = tuple(
        emb_table.T[8 * cg:8 * cg + 8, :_MAIN]
        .reshape(8, _FULL_TILES, 128)
        .transpose(1, 0, 2)
        .reshape(-1)
        for cg in range(4)
    )
    tail = emb_table[_MAIN:, :].reshape(-1)
    x = _sc_distance_arg(sidx, didx, tabs, tail)
    dist = _tc_arccosh(x.reshape(_B // 128, 128))
    return dist.reshape(_B)
